# Initial kernel scaffold; baseline (speedup 1.0000x reference)
import jax
import jax.numpy as jnp
from jax.experimental import pallas as pl


def kernel(C, keys, vals, temp):
    raise NotImplementedError("write your pallas kernel here")


# speedup vs baseline: 2.5502x; 2.5502x over previous
import jax
import jax.numpy as jnp
from jax.experimental import pallas as pl
from jax.experimental.pallas import tpu as pltpu


def _attn_kernel(scale_ref, c_ref, k_ref, v_ref, vhat_ref, alpha_ref):
    eps = 1e-8
    c = c_ref[0]
    k = k_ref[0]
    v = v_ref[0]

    c_n = jnp.sqrt(jnp.sum(c * c, axis=-1, keepdims=True))
    c_norm = c / jnp.maximum(c_n, eps)
    k_n = jnp.sqrt(jnp.sum(k * k, axis=-1, keepdims=True))
    k_norm = k / jnp.maximum(k_n, eps)

    sims = jax.lax.dot_general(
        c_norm, k_norm,
        dimension_numbers=(((1,), (1,)), ((), ())),
        preferred_element_type=jnp.float32,
    )
    s = sims * scale_ref[0, 0]
    m = jnp.max(s, axis=-1, keepdims=True)
    e = jnp.exp(s - m)
    alpha = e / jnp.sum(e, axis=-1, keepdims=True)

    vhat = jax.lax.dot_general(
        alpha, v,
        dimension_numbers=(((1,), (0,)), ((), ())),
        preferred_element_type=jnp.float32,
    )

    alpha_ref[0] = alpha
    vhat_ref[0] = vhat


@jax.jit
def kernel(C, keys, vals, temp):
    eps = 1e-8
    B, N, D = C.shape
    W = keys.shape[1]
    scale = (1.0 / (temp + eps)).reshape(1, 1).astype(jnp.float32)

    vhat, alpha = pl.pallas_call(
        _attn_kernel,
        grid=(B,),
        in_specs=[
            pl.BlockSpec(memory_space=pltpu.SMEM),
            pl.BlockSpec((1, N, D), lambda b: (b, 0, 0)),
            pl.BlockSpec((1, W, D), lambda b: (b, 0, 0)),
            pl.BlockSpec((1, W, D), lambda b: (b, 0, 0)),
        ],
        out_specs=[
            pl.BlockSpec((1, N, D), lambda b: (b, 0, 0)),
            pl.BlockSpec((1, N, W), lambda b: (b, 0, 0)),
        ],
        out_shape=[
            jax.ShapeDtypeStruct((B, N, D), jnp.float32),
            jax.ShapeDtypeStruct((B, N, W), jnp.float32),
        ],
    )(scale, C, keys, vals)
    return (vhat, alpha)


# speedup vs baseline: 3.4339x; 1.3465x over previous
import jax
import jax.numpy as jnp
from jax.experimental import pallas as pl
from jax.experimental.pallas import tpu as pltpu


def _attn_kernel(scale_ref, c_ref, k_ref, v_ref, vhat_ref, alpha_ref):
    eps = 1e-8
    c = c_ref[...]
    k = k_ref[...]

    c_n = jnp.sqrt(jnp.sum(c * c, axis=-1, keepdims=True))
    c_norm = c * (scale_ref[0, 0] / jnp.maximum(c_n, eps))
    k_n = jnp.sqrt(jnp.sum(k * k, axis=-1, keepdims=True))
    k_norm = k / jnp.maximum(k_n, eps)

    s = jax.lax.dot_general(
        c_norm, k_norm,
        dimension_numbers=(((1,), (1,)), ((), ())),
        preferred_element_type=jnp.float32,
    )
    e = jnp.exp(s)
    inv_z = 1.0 / jnp.sum(e, axis=-1, keepdims=True)

    r = jax.lax.dot_general(
        e, v_ref[...],
        dimension_numbers=(((1,), (0,)), ((), ())),
        preferred_element_type=jnp.float32,
    )

    alpha_ref[...] = e * inv_z
    vhat_ref[...] = r * inv_z


@jax.jit
def kernel(C, keys, vals, temp):
    eps = 1e-8
    B, N, D = C.shape
    W = keys.shape[1]
    scale = (1.0 / (temp + eps)).reshape(1, 1).astype(jnp.float32)

    vhat, alpha = pl.pallas_call(
        _attn_kernel,
        grid=(B,),
        in_specs=[
            pl.BlockSpec(memory_space=pltpu.SMEM),
            pl.BlockSpec((N, D), lambda b: (b, 0)),
            pl.BlockSpec((W, D), lambda b: (b, 0)),
            pl.BlockSpec((W, D), lambda b: (b, 0)),
        ],
        out_specs=[
            pl.BlockSpec((N, D), lambda b: (b, 0)),
            pl.BlockSpec((N, W), lambda b: (b, 0)),
        ],
        out_shape=[
            jax.ShapeDtypeStruct((B * N, D), jnp.float32),
            jax.ShapeDtypeStruct((B * N, W), jnp.float32),
        ],
    )(scale, C.reshape(B * N, D), keys.reshape(B * W, D), vals.reshape(B * W, D))
    return (vhat.reshape(B, N, D), alpha.reshape(B, N, W))


# speedup vs baseline: 3.4501x; 1.0047x over previous
import jax
import jax.numpy as jnp
from jax.experimental import pallas as pl
from jax.experimental.pallas import tpu as pltpu


def _attn_kernel(scale_ref, c_ref, k_ref, v_ref, vhat_ref, alpha_ref):
    eps = 1e-8
    c = c_ref[...]
    k = k_ref[...]

    c_n2 = jnp.sum(c * c, axis=-1, keepdims=True)
    c_norm = c * (scale_ref[0, 0] * jax.lax.rsqrt(jnp.maximum(c_n2, eps * eps)))
    k_n2 = jnp.sum(k * k, axis=-1, keepdims=True)
    k_norm = k * jax.lax.rsqrt(jnp.maximum(k_n2, eps * eps))

    s = jax.lax.dot_general(
        c_norm, k_norm,
        dimension_numbers=(((1,), (1,)), ((), ())),
        preferred_element_type=jnp.float32,
    )
    e = jnp.exp2(s)
    inv_z = 1.0 / jnp.sum(e, axis=-1, keepdims=True)

    r = jax.lax.dot_general(
        e, v_ref[...],
        dimension_numbers=(((1,), (0,)), ((), ())),
        preferred_element_type=jnp.float32,
    )

    alpha_ref[...] = e * inv_z
    vhat_ref[...] = r * inv_z


@jax.jit
def kernel(C, keys, vals, temp):
    eps = 1e-8
    B, N, D = C.shape
    W = keys.shape[1]
    scale = (1.4426950408889634 / (temp + eps)).reshape(1, 1).astype(jnp.float32)

    vhat, alpha = pl.pallas_call(
        _attn_kernel,
        grid=(B,),
        in_specs=[
            pl.BlockSpec(memory_space=pltpu.SMEM),
            pl.BlockSpec((N, D), lambda b: (b, 0)),
            pl.BlockSpec((W, D), lambda b: (b, 0)),
            pl.BlockSpec((W, D), lambda b: (b, 0)),
        ],
        out_specs=[
            pl.BlockSpec((N, D), lambda b: (b, 0)),
            pl.BlockSpec((N, W), lambda b: (b, 0)),
        ],
        out_shape=[
            jax.ShapeDtypeStruct((B * N, D), jnp.float32),
            jax.ShapeDtypeStruct((B * N, W), jnp.float32),
        ],
        compiler_params=pltpu.CompilerParams(
            dimension_semantics=("parallel",),
        ),
    )(scale, C.reshape(B * N, D), keys.reshape(B * W, D), vals.reshape(B * W, D))
    return (vhat.reshape(B, N, D), alpha.reshape(B, N, W))
